# Initial kernel scaffold; baseline (speedup 1.0000x reference)
import jax
import jax.numpy as jnp
from jax.experimental import pallas as pl


def kernel(local_preds, edge_index, edge_weight):
    raise NotImplementedError("write your pallas kernel here")


# speedup vs baseline: 6.2227x; 6.2227x over previous
import functools

import jax
import jax.numpy as jnp
from jax import lax
from jax.experimental import pallas as pl
from jax.experimental.pallas import tpu as pltpu
from jax.experimental.pallas import tpu_sc as plsc

N_NODES = 100000
N_EDGES = 1600000
D = 32
ALPHA = 0.1
NITER = 10

NW = 32
RPW = N_NODES // NW
B = 128
GROUPS = B // 16
PAD = 2 * B
CH_TOT = (N_EDGES + PAD) // B
MROW = 3 * B


def _spmm_iter(preds, lp_scaled, meta, params):
    mesh = plsc.VectorSubcoreMesh(core_axis_name="c", subcore_axis_name="s")

    @functools.partial(
        pl.kernel,
        mesh=mesh,
        compiler_params=pltpu.CompilerParams(use_tc_tiling_on_sc=False),
        out_type=jax.ShapeDtypeStruct((N_NODES * D,), jnp.float32),
        scratch_types=[
            pltpu.VMEM((RPW * D,), jnp.float32),
            pltpu.VMEM((B, D), jnp.float32),
            pltpu.VMEM((MROW,), jnp.int32),
            pltpu.VMEM((16,), jnp.int32),
            pltpu.SemaphoreType.DMA,
        ],
    )
    def k(preds_hbm, lp_hbm, meta_hbm, par_hbm, out_hbm,
          acc, msgs, mbuf, pv, sem):
        wid = lax.axis_index("s") * 2 + lax.axis_index("c")
        base = wid * (RPW * D)

        pltpu.sync_copy(par_hbm.at[pl.ds(wid * 16, 16)], pv)
        pvv = pv[pl.ds(0, 16)]
        cid0 = pvv[0]
        nch = pvv[1]
        s = pvv[2]
        e = pvv[3]

        pltpu.sync_copy(lp_hbm.at[pl.ds(base, RPW * D)], acc)

        iota16 = lax.iota(jnp.int32, 16)
        colref = mbuf.at[pl.ds(0, B)]

        def chunk_body(i, carry):
            cid = cid0 + i
            cbase = cid * B
            pltpu.sync_copy(meta_hbm.at[pl.ds(cid * MROW, MROW)], mbuf)
            pltpu.async_copy(preds_hbm.at[colref], msgs, sem).wait()
            for g in range(GROUPS):
                wg = lax.bitcast_convert_type(
                    mbuf[pl.ds(B + 16 * g, 16)], jnp.float32)
                rg = mbuf[pl.ds(2 * B + 16 * g, 16)]
                eg = iota16 + (cbase + 16 * g)
                wg = jnp.where((eg >= s) & (eg < e), wg, 0.0)
                for j in range(16):
                    wsc = wg[j]
                    off = rg[j] * D
                    for h in range(2):
                        x = msgs[16 * g + j, pl.ds(16 * h, 16)] * wsc
                        plsc.addupdate(acc.at[pl.ds(off + 16 * h, 16)], x)
            return carry

        lax.fori_loop(0, nch, chunk_body, 0)

        pltpu.sync_copy(acc, out_hbm.at[pl.ds(base, RPW * D)])

    return k(preds, lp_scaled, meta, params)


def kernel(local_preds, edge_index, edge_weight):
    row = edge_index[0].astype(jnp.int32)
    col = edge_index[1].astype(jnp.int32)
    w = ((1.0 - ALPHA) * edge_weight).astype(jnp.float32)

    row_s, col_s, w_s = lax.sort((row, col, w), num_keys=1)
    lrow_s = row_s % RPW

    zpad_i = jnp.zeros((PAD,), jnp.int32)
    col_p = jnp.concatenate([col_s, zpad_i])
    w_p = jnp.concatenate([w_s, jnp.zeros((PAD,), jnp.float32)])
    lrow_p = jnp.concatenate([lrow_s, zpad_i])

    meta = jnp.stack(
        [col_p.reshape(CH_TOT, B),
         jax.lax.bitcast_convert_type(w_p, jnp.int32).reshape(CH_TOT, B),
         lrow_p.reshape(CH_TOT, B)],
        axis=1,
    ).reshape(-1)

    bnd = jnp.searchsorted(
        row_s, jnp.arange(0, N_NODES + 1, RPW, dtype=jnp.int32)
    ).astype(jnp.int32)
    s = bnd[:-1]
    e = bnd[1:]
    cid0 = s // B
    nch = (e - cid0 * B + B - 1) // B
    zeros32 = jnp.zeros((NW,), jnp.int32)
    params = jnp.stack(
        [cid0, nch, s, e] + [zeros32] * 12, axis=1
    ).reshape(-1)

    lp_scaled = (ALPHA * local_preds).reshape(-1)
    preds = local_preds
    for _ in range(NITER):
        preds = _spmm_iter(preds, lp_scaled, meta, params).reshape(N_NODES, D)
    return preds
